# Initial kernel scaffold; baseline (speedup 1.0000x reference)
import jax
import jax.numpy as jnp
from jax.experimental import pallas as pl


def kernel(x, word_embeds, doc_weights, topic_embeds):
    raise NotImplementedError("write your pallas kernel here")


# speedup vs baseline: 1.5586x; 1.5586x over previous
import functools

import jax
import jax.numpy as jnp
from jax import lax
from jax.experimental import pallas as pl
from jax.experimental.pallas import tpu as pltpu
from jax.experimental.pallas import tpu_sc as plsc

WIN = 128


def _sc_gather(word_embeds, doc_weights, x0, x1):
    rows = x0.shape[1]
    embed = word_embeds.shape[1]
    topics = doc_weights.shape[1]
    mesh = plsc.VectorSubcoreMesh(core_axis_name="core", subcore_axis_name="subcore")

    @functools.partial(
        pl.kernel,
        out_type=(
            jax.ShapeDtypeStruct((rows, embed), jnp.float32),
            jax.ShapeDtypeStruct((rows, topics), jnp.float32),
        ),
        mesh=mesh,
        compiler_params=pltpu.CompilerParams(use_tc_tiling_on_sc=False),
    )
    def k(wtab_hbm, dtab_hbm, x0_hbm, x1_hbm, wout_hbm, dout_hbm):
        def body(i0_vmem, i1_vmem, wo_vmem, do_vmem):
            pltpu.sync_copy(wtab_hbm.at[i0_vmem.at[0]], wo_vmem)
            pltpu.sync_copy(dtab_hbm.at[i1_vmem.at[0]], do_vmem)

        pltpu.emit_pipeline(
            body,
            grid=(rows // WIN,),
            in_specs=[
                pl.BlockSpec((1, WIN), lambda i: (0, i)),
                pl.BlockSpec((1, WIN), lambda i: (0, i)),
            ],
            out_specs=[
                pl.BlockSpec((WIN, embed), lambda i: (i, 0)),
                pl.BlockSpec((WIN, topics), lambda i: (i, 0)),
            ],
            core_axis_name=("core", "subcore"),
            dimension_semantics=(pltpu.PARALLEL,),
        )(x0_hbm, x1_hbm, wout_hbm, dout_hbm)

    return k(word_embeds, doc_weights, x0, x1)


def _tc_finish(dw_g, word_g, te_t, gsum, gbc, seg_len):
    rows, topics = dw_g.shape
    embed = word_g.shape[1]
    samp = gsum.shape[0]
    rb = samp * seg_len
    nb = rows // rb

    def body(dw_ref, w_ref, te_ref, gs_ref, gb_ref, o_ref):
        e = jnp.exp(dw_ref[...])
        s = jnp.dot(gs_ref[...], e, preferred_element_type=jnp.float32)
        sinv_rows = jnp.dot(gb_ref[...], 1.0 / s,
                            preferred_element_type=jnp.float32)
        p = e * sinv_rows
        doc = jnp.dot(p, te_ref[...], preferred_element_type=jnp.float32)
        o_ref[...] = doc + w_ref[...]

    return pl.pallas_call(
        body,
        grid=(nb,),
        in_specs=[
            pl.BlockSpec((rb, topics), lambda i: (i, 0)),
            pl.BlockSpec((rb, embed), lambda i: (i, 0)),
            pl.BlockSpec((topics, embed), lambda i: (0, 0)),
            pl.BlockSpec((samp, rb), lambda i: (0, 0)),
            pl.BlockSpec((rb, samp), lambda i: (0, 0)),
        ],
        out_specs=pl.BlockSpec((rb, embed), lambda i: (i, 0)),
        out_shape=jax.ShapeDtypeStruct((rows, embed), jnp.float32),
    )(dw_g, word_g, te_t, gsum, gbc)


def kernel(x, word_embeds, doc_weights, topic_embeds):
    _, b, l = x.shape
    embed = word_embeds.shape[1]
    rows = b * l

    x0 = x[0].reshape(1, rows)
    x1 = x[1].reshape(1, rows)
    word_g, dw_g = _sc_gather(word_embeds, doc_weights, x0, x1)

    samp = 128
    rb = samp * l
    r_ids = jnp.arange(rb, dtype=jnp.int32) // l
    s_ids = jnp.arange(samp, dtype=jnp.int32)
    gsum = (r_ids[None, :] == s_ids[:, None]).astype(jnp.float32)
    gbc = gsum.T

    te_t = topic_embeds.T
    out = _tc_finish(dw_g, word_g, te_t, gsum, gbc, l)
    return out.reshape(b, l, embed)


# speedup vs baseline: 1.6406x; 1.0526x over previous
import functools

import jax
import jax.numpy as jnp
from jax import lax
from jax.experimental import pallas as pl
from jax.experimental.pallas import tpu as pltpu
from jax.experimental.pallas import tpu_sc as plsc

WIN = 128


def _sc_gather_word(word_embeds, x0):
    rows = x0.shape[1]
    embed = word_embeds.shape[1]
    mesh = plsc.VectorSubcoreMesh(core_axis_name="core", subcore_axis_name="subcore")

    @functools.partial(
        pl.kernel,
        out_type=jax.ShapeDtypeStruct((rows, embed), jnp.float32),
        mesh=mesh,
    )
    def k(wtab_hbm, x0_hbm, wout_hbm):
        def body(i0_vmem, wo_vmem):
            pltpu.sync_copy(wtab_hbm.at[i0_vmem.at[0]], wo_vmem)

        pltpu.emit_pipeline(
            body,
            grid=(rows // WIN,),
            in_specs=[pl.BlockSpec((1, WIN), lambda i: (0, i))],
            out_specs=[pl.BlockSpec((WIN, embed), lambda i: (i, 0))],
            core_axis_name=("core", "subcore"),
            dimension_semantics=(pltpu.PARALLEL,),
        )(x0_hbm, wout_hbm)

    return k(word_embeds, x0)


def _sc_gather_dw(doc_weights, x1):
    rows = x1.shape[1]
    topics = doc_weights.shape[1]
    mesh = plsc.VectorSubcoreMesh(core_axis_name="core", subcore_axis_name="subcore")

    @functools.partial(
        pl.kernel,
        out_type=jax.ShapeDtypeStruct((rows, topics), jnp.float32),
        mesh=mesh,
        compiler_params=pltpu.CompilerParams(use_tc_tiling_on_sc=False),
    )
    def k(dtab_hbm, x1_hbm, dout_hbm):
        def body(i1_vmem, do_vmem):
            pltpu.sync_copy(dtab_hbm.at[i1_vmem.at[0]], do_vmem)

        pltpu.emit_pipeline(
            body,
            grid=(rows // WIN,),
            in_specs=[pl.BlockSpec((1, WIN), lambda i: (0, i))],
            out_specs=[pl.BlockSpec((WIN, topics), lambda i: (i, 0))],
            core_axis_name=("core", "subcore"),
            dimension_semantics=(pltpu.PARALLEL,),
        )(x1_hbm, dout_hbm)

    return k(doc_weights, x1)


def _tc_finish(dw_g, word_g, te_t, gsum, gbc, seg_len):
    rows, topics = dw_g.shape
    embed = word_g.shape[1]
    samp = gsum.shape[0]
    rb = samp * seg_len
    nb = rows // rb

    def body(dw_ref, w_ref, te_ref, gs_ref, gb_ref, o_ref):
        e = jnp.exp(dw_ref[...])
        s = jnp.dot(gs_ref[...], e, preferred_element_type=jnp.float32)
        sinv_rows = jnp.dot(gb_ref[...], 1.0 / s,
                            preferred_element_type=jnp.float32)
        p = e * sinv_rows
        doc = jnp.dot(p, te_ref[...], preferred_element_type=jnp.float32)
        o_ref[...] = doc + w_ref[...]

    return pl.pallas_call(
        body,
        grid=(nb,),
        in_specs=[
            pl.BlockSpec((rb, topics), lambda i: (i, 0)),
            pl.BlockSpec((rb, embed), lambda i: (i, 0)),
            pl.BlockSpec((topics, embed), lambda i: (0, 0)),
            pl.BlockSpec((samp, rb), lambda i: (0, 0)),
            pl.BlockSpec((rb, samp), lambda i: (0, 0)),
        ],
        out_specs=pl.BlockSpec((rb, embed), lambda i: (i, 0)),
        out_shape=jax.ShapeDtypeStruct((rows, embed), jnp.float32),
    )(dw_g, word_g, te_t, gsum, gbc)


def kernel(x, word_embeds, doc_weights, topic_embeds):
    _, b, l = x.shape
    embed = word_embeds.shape[1]
    rows = b * l

    x0 = x[0].reshape(1, rows)
    x1 = x[1].reshape(1, rows)
    word_g = _sc_gather_word(word_embeds, x0)
    dw_g = _sc_gather_dw(doc_weights, x1)

    samp = 128
    rb = samp * l
    r_ids = jnp.arange(rb, dtype=jnp.int32) // l
    s_ids = jnp.arange(samp, dtype=jnp.int32)
    gsum = (r_ids[None, :] == s_ids[:, None]).astype(jnp.float32)
    gbc = gsum.T

    te_t = topic_embeds.T
    out = _tc_finish(dw_g, word_g, te_t, gsum, gbc, l)
    return out.reshape(b, l, embed)


# speedup vs baseline: 1.6726x; 1.0195x over previous
import functools

import jax
import jax.numpy as jnp
from jax import lax
from jax.experimental import pallas as pl
from jax.experimental.pallas import tpu as pltpu
from jax.experimental.pallas import tpu_sc as plsc

WIN = 128


def _sc_gather_word(word_embeds, x0):
    rows = x0.shape[1]
    embed = word_embeds.shape[1]
    mesh = plsc.VectorSubcoreMesh(core_axis_name="core", subcore_axis_name="subcore")

    @functools.partial(
        pl.kernel,
        out_type=jax.ShapeDtypeStruct((rows, embed), jnp.float32),
        mesh=mesh,
    )
    def k(wtab_hbm, x0_hbm, wout_hbm):
        def body(i0_vmem, wo_vmem):
            pltpu.sync_copy(wtab_hbm.at[i0_vmem.at[0]], wo_vmem)

        pltpu.emit_pipeline(
            body,
            grid=(rows // WIN,),
            in_specs=[pl.BlockSpec((1, WIN), lambda i: (0, i))],
            out_specs=[pl.BlockSpec((WIN, embed), lambda i: (i, 0))],
            core_axis_name=("core", "subcore"),
            dimension_semantics=(pltpu.PARALLEL,),
        )(x0_hbm, wout_hbm)

    return k(word_embeds, x0)


def _sc_gather_dw(doc_weights, x1):
    rows = x1.shape[1]
    topics = doc_weights.shape[1]
    mesh = plsc.VectorSubcoreMesh(core_axis_name="core", subcore_axis_name="subcore")

    @functools.partial(
        pl.kernel,
        out_type=jax.ShapeDtypeStruct((rows, topics), jnp.float32),
        mesh=mesh,
        compiler_params=pltpu.CompilerParams(use_tc_tiling_on_sc=False),
    )
    def k(dtab_hbm, x1_hbm, dout_hbm):
        def body(i1_vmem, do_vmem):
            pltpu.sync_copy(dtab_hbm.at[i1_vmem.at[0]], do_vmem)

        pltpu.emit_pipeline(
            body,
            grid=(rows // WIN,),
            in_specs=[pl.BlockSpec((1, WIN), lambda i: (0, i))],
            out_specs=[pl.BlockSpec((WIN, topics), lambda i: (i, 0))],
            core_axis_name=("core", "subcore"),
            dimension_semantics=(pltpu.PARALLEL,),
        )(x1_hbm, dout_hbm)

    return k(doc_weights, x1)


def _tc_finish(dw_p, word_g, te_t, samp, seg_len):
    topics = te_t.shape[0]
    rows, embed = word_g.shape
    rb = samp * seg_len
    nb = rows // rb

    def body(dw_ref, w_ref, te_ref, o_ref):
        e = jnp.exp(dw_ref[...])
        e3 = e.reshape(samp, seg_len, topics)
        s = jnp.sum(e3, axis=1, keepdims=True)
        p = (e3 / s).reshape(rb, topics)
        doc = jnp.dot(p, te_ref[...], preferred_element_type=jnp.float32)
        o_ref[...] = doc + w_ref[...]

    return pl.pallas_call(
        body,
        grid=(nb,),
        in_specs=[
            pl.BlockSpec((rb, topics), lambda i: (i, 0)),
            pl.BlockSpec((rb, embed), lambda i: (i, 0)),
            pl.BlockSpec((topics, embed), lambda i: (0, 0)),
        ],
        out_specs=pl.BlockSpec((rb, embed), lambda i: (i, 0)),
        out_shape=jax.ShapeDtypeStruct((rows, embed), jnp.float32),
    )(dw_p, word_g, te_t)


def kernel(x, word_embeds, doc_weights, topic_embeds):
    _, b, l = x.shape
    embed = word_embeds.shape[1]
    rows = b * l

    x0 = x[0].reshape(1, rows)
    x1 = x[1].reshape(1, rows)
    word_g = _sc_gather_word(word_embeds, x0)
    dw_g = _sc_gather_dw(doc_weights, x1)

    te_t = topic_embeds.T
    out = _tc_finish(dw_g, word_g, te_t, 128, l)
    return out.reshape(b, l, embed)


# speedup vs baseline: 2.6011x; 1.5551x over previous
import functools

import jax
import jax.numpy as jnp
from jax.experimental import pallas as pl
from jax.experimental.pallas import tpu as pltpu
from jax.experimental.pallas import tpu_sc as plsc

WIN = 128


def _sc_gather_word(word_embeds, x0):
    rows = x0.shape[1]
    embed = word_embeds.shape[1]
    mesh = plsc.VectorSubcoreMesh(core_axis_name="core", subcore_axis_name="subcore")

    @functools.partial(
        pl.kernel,
        out_type=jax.ShapeDtypeStruct((rows, embed), jnp.float32),
        mesh=mesh,
    )
    def k(wtab_hbm, x0_hbm, wout_hbm):
        def body(i0_vmem, wo_vmem):
            pltpu.sync_copy(wtab_hbm.at[i0_vmem.at[0]], wo_vmem)

        pltpu.emit_pipeline(
            body,
            grid=(rows // WIN,),
            in_specs=[pl.BlockSpec((1, WIN), lambda i: (0, i))],
            out_specs=[pl.BlockSpec((WIN, embed), lambda i: (i, 0))],
            core_axis_name=("core", "subcore"),
            dimension_semantics=(pltpu.PARALLEL,),
        )(x0_hbm, wout_hbm)

    return k(word_embeds, x0)


def _sc_gather_dw(doc_weights, x1):
    rows = x1.shape[1]
    topics = doc_weights.shape[1]
    mesh = plsc.VectorSubcoreMesh(core_axis_name="core", subcore_axis_name="subcore")

    @functools.partial(
        pl.kernel,
        out_type=jax.ShapeDtypeStruct((rows, 128), jnp.float32),
        mesh=mesh,
        scratch_types=[pltpu.VMEM((WIN, topics), jnp.float32)],
        compiler_params=pltpu.CompilerParams(use_tc_tiling_on_sc=False),
    )
    def k(dtab_hbm, x1_hbm, dout_hbm, tmp_vmem):
        def body(i1_vmem, do_vmem):
            pltpu.sync_copy(dtab_hbm.at[i1_vmem.at[0]], tmp_vmem)

            @pl.loop(0, WIN)
            def _(r):
                do_vmem[r, pl.ds(0, 16)] = tmp_vmem[r, pl.ds(0, 16)]
                do_vmem[r, pl.ds(16, 16)] = tmp_vmem[r, pl.ds(16, 16)]

        pltpu.emit_pipeline(
            body,
            grid=(rows // WIN,),
            in_specs=[pl.BlockSpec((1, WIN), lambda i: (0, i))],
            out_specs=[pl.BlockSpec((WIN, 128), lambda i: (i, 0))],
            core_axis_name=("core", "subcore"),
            dimension_semantics=(pltpu.PARALLEL,),
        )(x1_hbm, dout_hbm)

    return k(doc_weights, x1)


def _tc_finish(dw3, word3, te_t, bs):
    l, b, _ = dw3.shape
    topics = te_t.shape[0]
    embed = word3.shape[2]
    nb = b // bs

    def body(dw_ref, w_ref, te_ref, o_ref):
        e = jnp.exp(dw_ref[:, :, :topics])
        s = jnp.sum(e, axis=0, keepdims=True)
        p = (e / s).reshape(l * bs, topics)
        doc = jnp.dot(p, te_ref[...], preferred_element_type=jnp.float32)
        o_ref[...] = (doc + w_ref[...].reshape(l * bs, embed)).reshape(
            l, bs, embed)

    return pl.pallas_call(
        body,
        grid=(nb,),
        in_specs=[
            pl.BlockSpec((l, bs, 128), lambda i: (0, i, 0)),
            pl.BlockSpec((l, bs, embed), lambda i: (0, i, 0)),
            pl.BlockSpec((topics, embed), lambda i: (0, 0)),
        ],
        out_specs=pl.BlockSpec((l, bs, embed), lambda i: (0, i, 0)),
        out_shape=jax.ShapeDtypeStruct((l, b, embed), jnp.float32),
    )(dw3, word3, te_t)


def kernel(x, word_embeds, doc_weights, topic_embeds):
    _, b, l = x.shape
    embed = word_embeds.shape[1]
    rows = b * l

    x0 = x[0].transpose(1, 0).reshape(1, rows)
    x1 = x[1].transpose(1, 0).reshape(1, rows)
    word_g = _sc_gather_word(word_embeds, x0)
    dw_g = _sc_gather_dw(doc_weights, x1)

    word3 = word_g.reshape(l, b, embed)
    dw3 = dw_g.reshape(l, b, 128)
    te_t = topic_embeds.T
    out3 = _tc_finish(dw3, word3, te_t, 256)
    return jnp.transpose(out3, (1, 0, 2))
